# baseline (speedup 1.0000x reference)
import functools

import jax
import jax.numpy as jnp
from jax import lax
from jax.experimental import pallas as pl
from jax.experimental.pallas import tpu as pltpu
from jax.experimental.pallas import tpu_sc as plsc

NUM_CARDS = 1000000
EMBED_DIM = 32
BATCH = 16384
HIST = 200

NW = 32
BB = BATCH // 128
BB_PER_W = BB // NW
NBUF = 4


def _gather_kernel(idsT_hbm, table_hbm, out_hbm, idx_v, rows_v, sem_i, sem_g, sem_o):
    wid = lax.axis_index("s") * 2 + lax.axis_index("c")
    bb0 = wid * BB_PER_W

    def idx_copy(h, u, s):
        return pltpu.make_async_copy(
            idsT_hbm.at[h, pl.ds((bb0 + u) * 128, 128)], idx_v.at[s], sem_i.at[s])

    def gather_copy(s):
        return pltpu.make_async_copy(
            table_hbm.at[idx_v.at[s]], rows_v.at[s], sem_g.at[s])

    def out_copy(h, u, s):
        return pltpu.make_async_copy(
            rows_v.at[s], out_hbm.at[h, pl.ds((bb0 + u) * 128, 128)], sem_o.at[s])

    idx_copy(0, 0, 0).start()

    def body(h, carry):
        for u in range(NBUF):
            @pl.when(h >= 1)
            def _():
                out_copy(h - 1, u, u).wait()

            idx_copy(h, u, u).wait()
            gather_copy(u).start()

            prev = (u - 1) % NBUF
            if u > 0:
                gather_copy(prev).wait()
                out_copy(h, u - 1, prev).start()
            else:
                @pl.when(h >= 1)
                def _():
                    gather_copy(prev).wait()
                    out_copy(h - 1, NBUF - 1, prev).start()

            if u < NBUF - 1:
                idx_copy(h, u + 1, u + 1).start()
            else:
                @pl.when(h < HIST - 1)
                def _():
                    idx_copy(h + 1, 0, 0).start()
        return carry

    lax.fori_loop(0, HIST, body, 0)

    gather_copy(NBUF - 1).wait()
    out_copy(HIST - 1, NBUF - 1, NBUF - 1).start()
    for u in range(NBUF):
        out_copy(HIST - 1, u, u).wait()


@jax.jit
def _embed(idsT, table):
    fn = functools.partial(
        pl.kernel,
        out_type=jax.ShapeDtypeStruct((HIST, BATCH, EMBED_DIM), jnp.float32),
        mesh=plsc.VectorSubcoreMesh(core_axis_name="c", subcore_axis_name="s"),
        scratch_types=[
            pltpu.VMEM((NBUF, 128), jnp.int32),
            pltpu.VMEM((NBUF, 128, EMBED_DIM), jnp.float32),
            pltpu.SemaphoreType.DMA((NBUF,)),
            pltpu.SemaphoreType.DMA((NBUF,)),
            pltpu.SemaphoreType.DMA((NBUF,)),
        ],
        compiler_params=pltpu.CompilerParams(use_tc_tiling_on_sc=False),
    )(_gather_kernel)
    return fn(idsT, table)


def kernel(ids, table):
    idsT = ids.astype(jnp.int32).T
    out_h = _embed(idsT, table)
    return jnp.transpose(out_h, (1, 0, 2))
